# baseline (speedup 1.0000x reference)
import functools

import jax
import jax.numpy as jnp
from jax.experimental import pallas as pl
from jax.experimental.pallas import tpu as pltpu


_BM = 400


def _gcn_kernel(bm, x_ref, w_ref, wl_ref, b_ref, a1_ref, a2_ref, out_ref, s_ref, l_ref):
    i = pl.program_id(0)
    h = bm // 2

    @pl.when(i == 0)
    def _():
        x = x_ref[...]
        s_ref[...] = jnp.dot(x, w_ref[...], preferred_element_type=jnp.float32)
        l_ref[...] = (
            jnp.dot(x, wl_ref[...], preferred_element_type=jnp.float32)
            + b_ref[...]
        )

    s = s_ref[...]
    out_ref[pl.ds(0, h), :] = (
        jnp.dot(a1_ref[...], s, preferred_element_type=jnp.float32)
        + l_ref[pl.ds(i * bm, h), :]
    )
    out_ref[pl.ds(h, h), :] = (
        jnp.dot(a2_ref[...], s, preferred_element_type=jnp.float32)
        + l_ref[pl.ds(i * bm + h, h), :]
    )


def kernel(inputs, adj_mat, weight, loop_weight, bias):
    n, d_in = inputs.shape
    d_out = weight.shape[1]
    bm = _BM
    assert n % bm == 0
    h = bm // 2
    grid_m = n // bm

    bias2d = bias.reshape(1, d_out)

    return pl.pallas_call(
        functools.partial(_gcn_kernel, bm),
        grid=(grid_m,),
        in_specs=[
            pl.BlockSpec((n, d_in), lambda i: (0, 0)),
            pl.BlockSpec((d_in, d_out), lambda i: (0, 0)),
            pl.BlockSpec((d_in, d_out), lambda i: (0, 0)),
            pl.BlockSpec((1, d_out), lambda i: (0, 0)),
            pl.BlockSpec((h, n), lambda i: (2 * i, 0)),
            pl.BlockSpec((h, n), lambda i: (2 * i + 1, 0)),
        ],
        out_specs=pl.BlockSpec((bm, d_out), lambda i: (i, 0)),
        out_shape=jax.ShapeDtypeStruct((n, d_out), jnp.float32),
        scratch_shapes=[
            pltpu.VMEM((n, d_out), jnp.float32),
            pltpu.VMEM((n, d_out), jnp.float32),
        ],
    )(inputs, weight, loop_weight, bias2d, adj_mat, adj_mat)


# speedup vs baseline: 1.0098x; 1.0098x over previous
import functools

import jax
import jax.numpy as jnp
from jax.experimental import pallas as pl
from jax.experimental.pallas import tpu as pltpu


_BM = 320


def _gcn_kernel(bm, n, x_ref, w_ref, wl_ref, b_ref, adj_ref, out_ref, s_ref, l_ref):
    i = pl.program_id(0)

    @pl.when(i == 0)
    def _():
        x = x_ref[...]
        s_ref[...] = jnp.dot(x, w_ref[...], preferred_element_type=jnp.float32)
        l_ref[pl.ds(0, n), :] = (
            jnp.dot(x, wl_ref[...], preferred_element_type=jnp.float32)
            + b_ref[...]
        )

    out_ref[...] = (
        jnp.dot(adj_ref[...], s_ref[...], preferred_element_type=jnp.float32)
        + l_ref[pl.ds(i * bm, bm), :]
    )


def kernel(inputs, adj_mat, weight, loop_weight, bias):
    n, d_in = inputs.shape
    d_out = weight.shape[1]
    bm = max(8, min(_BM, ((n + 7) // 8) * 8))
    grid_m = (n + bm - 1) // bm

    bias2d = bias.reshape(1, d_out)

    return pl.pallas_call(
        functools.partial(_gcn_kernel, bm, n),
        grid=(grid_m,),
        in_specs=[
            pl.BlockSpec((n, d_in), lambda i: (0, 0)),
            pl.BlockSpec((d_in, d_out), lambda i: (0, 0)),
            pl.BlockSpec((d_in, d_out), lambda i: (0, 0)),
            pl.BlockSpec((1, d_out), lambda i: (0, 0)),
            pl.BlockSpec((bm, n), lambda i: (i, 0)),
        ],
        out_specs=pl.BlockSpec((bm, d_out), lambda i: (i, 0)),
        out_shape=jax.ShapeDtypeStruct((n, d_out), jnp.float32),
        scratch_shapes=[
            pltpu.VMEM((n, d_out), jnp.float32),
            pltpu.VMEM((grid_m * bm, d_out), jnp.float32),
        ],
    )(inputs, weight, loop_weight, bias2d, adj_mat)


# speedup vs baseline: 1.0184x; 1.0085x over previous
import functools

import jax
import jax.numpy as jnp
from jax.experimental import pallas as pl
from jax.experimental.pallas import tpu as pltpu


_BM = 288


def _gcn_kernel(bm, n, x_ref, w_ref, wl_ref, b_ref, adj_ref, out_ref, s_ref, l_ref):
    i = pl.program_id(0)

    @pl.when(i == 0)
    def _():
        x = x_ref[...]
        s_ref[...] = jnp.dot(x, w_ref[...], preferred_element_type=jnp.float32)
        l_ref[pl.ds(0, n), :] = (
            jnp.dot(x, wl_ref[...], preferred_element_type=jnp.float32)
            + b_ref[...]
        )

    out_ref[...] = (
        jnp.dot(adj_ref[...], s_ref[...], preferred_element_type=jnp.float32)
        + l_ref[pl.ds(i * bm, bm), :]
    )


def kernel(inputs, adj_mat, weight, loop_weight, bias):
    n, d_in = inputs.shape
    d_out = weight.shape[1]
    bm = max(8, min(_BM, ((n + 7) // 8) * 8))
    grid_m = (n + bm - 1) // bm

    bias2d = bias.reshape(1, d_out)

    return pl.pallas_call(
        functools.partial(_gcn_kernel, bm, n),
        grid=(grid_m,),
        in_specs=[
            pl.BlockSpec((n, d_in), lambda i: (0, 0)),
            pl.BlockSpec((d_in, d_out), lambda i: (0, 0)),
            pl.BlockSpec((d_in, d_out), lambda i: (0, 0)),
            pl.BlockSpec((1, d_out), lambda i: (0, 0)),
            pl.BlockSpec((bm, n), lambda i: (i, 0)),
        ],
        out_specs=pl.BlockSpec((bm, d_out), lambda i: (i, 0)),
        out_shape=jax.ShapeDtypeStruct((n, d_out), jnp.float32),
        scratch_shapes=[
            pltpu.VMEM((n, d_out), jnp.float32),
            pltpu.VMEM((grid_m * bm, d_out), jnp.float32),
        ],
    )(inputs, weight, loop_weight, bias2d, adj_mat)


# speedup vs baseline: 1.0190x; 1.0006x over previous
import functools

import jax
import jax.numpy as jnp
from jax.experimental import pallas as pl
from jax.experimental.pallas import tpu as pltpu


_BM = 272


def _gcn_kernel(bm, n, x_ref, w_ref, wl_ref, b_ref, adj_ref, out_ref, s_ref, l_ref):
    i = pl.program_id(0)

    @pl.when(i == 0)
    def _():
        x = x_ref[...]
        s_ref[...] = jnp.dot(x, w_ref[...], preferred_element_type=jnp.float32)
        l_ref[pl.ds(0, n), :] = (
            jnp.dot(x, wl_ref[...], preferred_element_type=jnp.float32)
            + b_ref[...]
        )

    out_ref[...] = (
        jnp.dot(adj_ref[...], s_ref[...], preferred_element_type=jnp.float32)
        + l_ref[pl.ds(i * bm, bm), :]
    )


def kernel(inputs, adj_mat, weight, loop_weight, bias):
    n, d_in = inputs.shape
    d_out = weight.shape[1]
    bm = max(8, min(_BM, ((n + 7) // 8) * 8))
    grid_m = (n + bm - 1) // bm

    bias2d = bias.reshape(1, d_out)

    return pl.pallas_call(
        functools.partial(_gcn_kernel, bm, n),
        grid=(grid_m,),
        in_specs=[
            pl.BlockSpec((n, d_in), lambda i: (0, 0)),
            pl.BlockSpec((d_in, d_out), lambda i: (0, 0)),
            pl.BlockSpec((d_in, d_out), lambda i: (0, 0)),
            pl.BlockSpec((1, d_out), lambda i: (0, 0)),
            pl.BlockSpec((bm, n), lambda i: (i, 0)),
        ],
        out_specs=pl.BlockSpec((bm, d_out), lambda i: (i, 0)),
        out_shape=jax.ShapeDtypeStruct((n, d_out), jnp.float32),
        scratch_shapes=[
            pltpu.VMEM((n, d_out), jnp.float32),
            pltpu.VMEM((grid_m * bm, d_out), jnp.float32),
        ],
    )(inputs, weight, loop_weight, bias2d, adj_mat)
